# baseline (speedup 1.0000x reference)
import functools

import jax
import jax.numpy as jnp
from jax.experimental import pallas as pl
from jax.experimental.pallas import tpu as pltpu
from jax.experimental.layout import Format, Layout

B, S, D, C = 1024, 200, 128, 49
BB = 32


def _body(s_ref, e_ref, x_ref, w1_ref, b1_ref, w2_ref, b2_ref,
          out1_ref, out2_ref, xmod_ref):
    g = pl.program_id(0)
    iota = jax.lax.broadcasted_iota(jnp.int32, (S, 1), 0)
    for j in range(BB):
        i = g * BB + j
        s = s_ref[i]
        e = e_ref[i]
        xj = x_ref[j]
        vs = x_ref[j, pl.ds(s, 1), :]
        ve = x_ref[j, pl.ds(e, 1), :]
        v = 0.5 * (vs + ve)
        coef = ((iota == s) | (iota == e)).astype(jnp.float32)
        xmod_ref[pl.ds(j * S, S), :] = xj + coef * v
    xmod = xmod_ref[...]
    p1 = jnp.dot(xmod, w1_ref[...], preferred_element_type=jnp.float32) \
        + b1_ref[...]
    p2 = jnp.dot(xmod, w2_ref[...], preferred_element_type=jnp.float32) \
        + b2_ref[...]
    o1 = 0.5 * jnp.tanh(0.5 * p1) + 0.5
    o2 = 0.5 * jnp.tanh(0.5 * p2) + 0.5
    out1_ref[...] = o1.reshape(BB, S, C)
    out2_ref[...] = o2.reshape(BB, S, C)


def _kernel_impl(x_lstm, position_s, position_e, W1, b1, W2, b2):
    b1r = b1.reshape(1, C)
    b2r = b2.reshape(1, C)
    pos_s = position_s.astype(jnp.int32)
    pos_e = position_e.astype(jnp.int32)
    grid = B // BB
    out1, out2 = pl.pallas_call(
        _body,
        grid_spec=pltpu.PrefetchScalarGridSpec(
            num_scalar_prefetch=2,
            grid=(grid,),
            in_specs=[
                pl.BlockSpec((BB, S, D), lambda g, *_: (g, 0, 0)),
                pl.BlockSpec((D, C), lambda g, *_: (0, 0)),
                pl.BlockSpec((1, C), lambda g, *_: (0, 0)),
                pl.BlockSpec((D, C), lambda g, *_: (0, 0)),
                pl.BlockSpec((1, C), lambda g, *_: (0, 0)),
            ],
            out_specs=[
                pl.BlockSpec((BB, S, C), lambda g, *_: (g, 0, 0)),
                pl.BlockSpec((BB, S, C), lambda g, *_: (g, 0, 0)),
            ],
            scratch_shapes=[pltpu.VMEM((BB * S, D), jnp.float32)],
        ),
        out_shape=[
            jax.ShapeDtypeStruct((B, S, C), jnp.float32),
            jax.ShapeDtypeStruct((B, S, C), jnp.float32),
        ],
        compiler_params=pltpu.CompilerParams(
            dimension_semantics=("parallel",),
        ),
    )(pos_s, pos_e, x_lstm, W1, b1r, W2, b2r)
    return (out1, out2)


_jitted = None


def kernel(x_lstm, position_s, position_e, W1, b1, W2, b2):
    global _jitted
    if _jitted is None:
        sharding = jax.sharding.SingleDeviceSharding(jax.devices()[0])
        fmt = Format(Layout(major_to_minor=(0, 1, 2)), sharding)
        _jitted = jax.jit(_kernel_impl, out_shardings=(fmt, fmt))
    return _jitted(x_lstm, position_s, position_e, W1, b1, W2, b2)


# speedup vs baseline: 1.3790x; 1.3790x over previous
import jax
import jax.numpy as jnp
from jax.experimental import pallas as pl
from jax.experimental.pallas import tpu as pltpu

B, S, D, C = 1024, 200, 128, 49
BB = 32


def _body(s_ref, e_ref, x_ref, w1_ref, b1_ref, w2_ref, b2_ref,
          out1_ref, out2_ref, xmod_ref):
    g = pl.program_id(0)
    iota = jax.lax.broadcasted_iota(jnp.int32, (S, 1), 0)
    for j in range(BB):
        i = g * BB + j
        s = s_ref[i]
        e = e_ref[i]
        xj = x_ref[j]
        vs = x_ref[j, pl.ds(s, 1), :]
        ve = x_ref[j, pl.ds(e, 1), :]
        v = 0.5 * (vs + ve)
        coef = ((iota == s) | (iota == e)).astype(jnp.float32)
        xmod_ref[pl.ds(j * S, S), :] = xj + coef * v
    xmod = xmod_ref[...]
    dn = (((0,), (1,)), ((), ()))
    p1 = jax.lax.dot_general(w1_ref[...], xmod, dn,
                             preferred_element_type=jnp.float32) + b1_ref[...]
    p2 = jax.lax.dot_general(w2_ref[...], xmod, dn,
                             preferred_element_type=jnp.float32) + b2_ref[...]
    o1 = 0.5 * jnp.tanh(0.5 * p1) + 0.5
    o2 = 0.5 * jnp.tanh(0.5 * p2) + 0.5
    out1_ref[...] = o1.reshape(C, BB, S)
    out2_ref[...] = o2.reshape(C, BB, S)


def _kernel_impl(x_lstm, position_s, position_e, W1, b1, W2, b2):
    b1r = b1.reshape(C, 1)
    b2r = b2.reshape(C, 1)
    pos_s = position_s.astype(jnp.int32)
    pos_e = position_e.astype(jnp.int32)
    grid = B // BB
    out1, out2 = pl.pallas_call(
        _body,
        grid_spec=pltpu.PrefetchScalarGridSpec(
            num_scalar_prefetch=2,
            grid=(grid,),
            in_specs=[
                pl.BlockSpec((BB, S, D), lambda g, *_: (g, 0, 0)),
                pl.BlockSpec((D, C), lambda g, *_: (0, 0)),
                pl.BlockSpec((C, 1), lambda g, *_: (0, 0)),
                pl.BlockSpec((D, C), lambda g, *_: (0, 0)),
                pl.BlockSpec((C, 1), lambda g, *_: (0, 0)),
            ],
            out_specs=[
                pl.BlockSpec((C, BB, S), lambda g, *_: (0, g, 0)),
                pl.BlockSpec((C, BB, S), lambda g, *_: (0, g, 0)),
            ],
            scratch_shapes=[pltpu.VMEM((BB * S, D), jnp.float32)],
        ),
        out_shape=[
            jax.ShapeDtypeStruct((C, B, S), jnp.float32),
            jax.ShapeDtypeStruct((C, B, S), jnp.float32),
        ],
        compiler_params=pltpu.CompilerParams(
            dimension_semantics=("parallel",),
        ),
    )(pos_s, pos_e, x_lstm, W1, b1r, W2, b2r)
    return (jnp.transpose(out1, (1, 2, 0)), jnp.transpose(out2, (1, 2, 0)))


kernel = jax.jit(_kernel_impl)


# speedup vs baseline: 2.2082x; 1.6013x over previous
import jax
import jax.numpy as jnp
from jax.experimental import pallas as pl
from jax.experimental.pallas import tpu as pltpu

B, S, D, C = 1024, 200, 128, 49
BB = 128
G = B // BB
NS = 5
SS = S // NS


def _body(ss_ref, se_ref, vs_ref, ve_ref, x_ref, w1_ref, b1_ref,
          w2_ref, b2_ref, out1_ref, out2_ref, d1_ref, d2_ref, v_ref):
    g = pl.program_id(0)
    sb = pl.program_id(1)

    @pl.when(sb == 0)
    def _():
        for j in range(BB):
            i = g * BB + j
            sj = ss_ref[i]
            ej = se_ref[i]
            v_ref[pl.ds(j, 1), :] = 0.5 * (x_ref[j, pl.ds(sj, 1), :]
                                           + x_ref[j, pl.ds(ej, 1), :])
        dn = (((0,), (1,)), ((), ()))
        d1_ref[...] = jax.lax.dot_general(
            w1_ref[...], v_ref[...], dn, preferred_element_type=jnp.float32)
        d2_ref[...] = jax.lax.dot_general(
            w2_ref[...], v_ref[...], dn, preferred_element_type=jnp.float32)

    pos_s = vs_ref[0, 0, :].reshape(1, BB)
    pos_e = ve_ref[0, 0, :].reshape(1, BB)
    iota_s = sb * SS + jax.lax.broadcasted_iota(jnp.int32, (SS, BB), 0)
    coef = ((iota_s == pos_s) | (iota_s == pos_e)).astype(jnp.float32)

    xc = x_ref[:, pl.ds(sb * SS, SS), :]
    xt = jnp.swapaxes(xc, 0, 1).reshape(SS * BB, D)
    dn = (((0,), (1,)), ((), ()))
    for w_ref, b_ref, d_ref, out_ref in (
            (w1_ref, b1_ref, d1_ref, out1_ref),
            (w2_ref, b2_ref, d2_ref, out2_ref)):
        p = jax.lax.dot_general(w_ref[...], xt, dn,
                                preferred_element_type=jnp.float32)
        p3 = p.reshape(C, SS, BB) + b_ref[...].reshape(C, 1, 1)
        p3 = p3 + d_ref[...].reshape(C, 1, BB) * coef.reshape(1, SS, BB)
        out_ref[...] = 0.5 * jnp.tanh(0.5 * p3) + 0.5


def _kernel_impl(x_lstm, position_s, position_e, W1, b1, W2, b2):
    b1r = b1.reshape(C, 1)
    b2r = b2.reshape(C, 1)
    pos_s = position_s.astype(jnp.int32)
    pos_e = position_e.astype(jnp.int32)
    pos_s3 = pos_s.reshape(G, 1, BB)
    pos_e3 = pos_e.reshape(G, 1, BB)
    out1, out2 = pl.pallas_call(
        _body,
        grid_spec=pltpu.PrefetchScalarGridSpec(
            num_scalar_prefetch=2,
            grid=(G, NS),
            in_specs=[
                pl.BlockSpec((1, 1, BB), lambda g, sb, *_: (g, 0, 0)),
                pl.BlockSpec((1, 1, BB), lambda g, sb, *_: (g, 0, 0)),
                pl.BlockSpec((BB, S, D), lambda g, sb, *_: (g, 0, 0)),
                pl.BlockSpec((D, C), lambda g, sb, *_: (0, 0)),
                pl.BlockSpec((C, 1), lambda g, sb, *_: (0, 0)),
                pl.BlockSpec((D, C), lambda g, sb, *_: (0, 0)),
                pl.BlockSpec((C, 1), lambda g, sb, *_: (0, 0)),
            ],
            out_specs=[
                pl.BlockSpec((C, SS, BB), lambda g, sb, *_: (0, sb, g)),
                pl.BlockSpec((C, SS, BB), lambda g, sb, *_: (0, sb, g)),
            ],
            scratch_shapes=[
                pltpu.VMEM((C, BB), jnp.float32),
                pltpu.VMEM((C, BB), jnp.float32),
                pltpu.VMEM((BB, D), jnp.float32),
            ],
        ),
        out_shape=[
            jax.ShapeDtypeStruct((C, S, B), jnp.float32),
            jax.ShapeDtypeStruct((C, S, B), jnp.float32),
        ],
        compiler_params=pltpu.CompilerParams(
            dimension_semantics=("parallel", "arbitrary"),
        ),
    )(pos_s, pos_e, pos_s3, pos_e3, x_lstm, W1, b1r, W2, b2r)
    return (jnp.transpose(out1, (2, 1, 0)), jnp.transpose(out2, (2, 1, 0)))


kernel = jax.jit(_kernel_impl)


# speedup vs baseline: 2.2935x; 1.0387x over previous
import jax
import jax.numpy as jnp
from jax.experimental import pallas as pl
from jax.experimental.pallas import tpu as pltpu
from jax.experimental.pallas import tpu_sc as plsc

B, S, D, C = 1024, 200, 128, 49
BB = 128
G = B // BB
NS = 5
SS = S // NS
_GATHER_WINDOW = 128


def _sc_gather(x2d, indices):
    n_idx = indices.shape[0]
    indices = indices.reshape(1, n_idx)
    mesh = plsc.VectorSubcoreMesh(core_axis_name="core",
                                  subcore_axis_name="subcore")

    @pl.kernel(out_type=jax.ShapeDtypeStruct((n_idx, D), x2d.dtype),
               mesh=mesh)
    def gather_kernel(x_hbm, i_hbm, o_hbm):
        def body(i_vmem, o_vmem):
            pltpu.sync_copy(x_hbm.at[i_vmem.at[0]], o_vmem)

        pltpu.emit_pipeline(
            body,
            grid=(n_idx // _GATHER_WINDOW,),
            in_specs=[pl.BlockSpec((1, _GATHER_WINDOW),
                                   index_map=lambda i: (0, i))],
            out_specs=[pl.BlockSpec((_GATHER_WINDOW, D),
                                    index_map=lambda i: (i, 0))],
            core_axis_name="subcore",
            dimension_semantics=(pltpu.PARALLEL,),
        )(i_hbm, o_hbm)

    return gather_kernel(x2d, indices)


def _body(vs_ref, ve_ref, x_ref, w1_ref, b1_ref, w2_ref, b2_ref,
          ps_ref, pe_ref, out1_ref, out2_ref):
    sb = pl.program_id(1)
    dn = (((0,), (1,)), ((), ()))
    v = 0.5 * (vs_ref[...] + ve_ref[...])

    pos_s = ps_ref[0, 0, :].reshape(1, BB)
    pos_e = pe_ref[0, 0, :].reshape(1, BB)
    iota_s = sb * SS + jax.lax.broadcasted_iota(jnp.int32, (SS, BB), 0)
    coef = ((iota_s == pos_s) | (iota_s == pos_e)).astype(jnp.float32)

    xt = jnp.swapaxes(x_ref[...], 0, 1).reshape(SS * BB, D)
    for w_ref, b_ref, out_ref in ((w1_ref, b1_ref, out1_ref),
                                  (w2_ref, b2_ref, out2_ref)):
        p = jax.lax.dot_general(w_ref[...], xt, dn,
                                preferred_element_type=jnp.float32)
        delta = jax.lax.dot_general(w_ref[...], v, dn,
                                    preferred_element_type=jnp.float32)
        p3 = p.reshape(C, SS, BB) + b_ref[...].reshape(C, 1, 1)
        p3 = p3 + delta.reshape(C, 1, BB) * coef.reshape(1, SS, BB)
        out_ref[...] = 0.5 * jnp.tanh(0.5 * p3) + 0.5


def _kernel_impl(x_lstm, position_s, position_e, W1, b1, W2, b2):
    b1r = b1.reshape(C, 1)
    b2r = b2.reshape(C, 1)
    pos_s = position_s.astype(jnp.int32)
    pos_e = position_e.astype(jnp.int32)

    x2d = x_lstm.reshape(B * S, D)
    row_ids = jnp.arange(B, dtype=jnp.int32) * S
    gathered = _sc_gather(x2d, jnp.concatenate([row_ids + pos_s,
                                                row_ids + pos_e]))

    pos_s3 = pos_s.reshape(G, 1, BB)
    pos_e3 = pos_e.reshape(G, 1, BB)
    out1, out2 = pl.pallas_call(
        _body,
        grid=(G, NS),
        in_specs=[
            pl.BlockSpec((BB, D), lambda g, sb: (g, 0)),
            pl.BlockSpec((BB, D), lambda g, sb: (G + g, 0)),
            pl.BlockSpec((BB, SS, D), lambda g, sb: (g, sb, 0)),
            pl.BlockSpec((D, C), lambda g, sb: (0, 0)),
            pl.BlockSpec((C, 1), lambda g, sb: (0, 0)),
            pl.BlockSpec((D, C), lambda g, sb: (0, 0)),
            pl.BlockSpec((C, 1), lambda g, sb: (0, 0)),
            pl.BlockSpec((1, 1, BB), lambda g, sb: (g, 0, 0)),
            pl.BlockSpec((1, 1, BB), lambda g, sb: (g, 0, 0)),
        ],
        out_specs=[
            pl.BlockSpec((C, SS, BB), lambda g, sb: (0, sb, g)),
            pl.BlockSpec((C, SS, BB), lambda g, sb: (0, sb, g)),
        ],
        out_shape=[
            jax.ShapeDtypeStruct((C, S, B), jnp.float32),
            jax.ShapeDtypeStruct((C, S, B), jnp.float32),
        ],
        compiler_params=pltpu.CompilerParams(
            dimension_semantics=("parallel", "arbitrary"),
        ),
    )(gathered, gathered, x_lstm, W1, b1r, W2, b2r, pos_s3, pos_e3)
    return (jnp.transpose(out1, (2, 1, 0)), jnp.transpose(out2, (2, 1, 0)))


kernel = jax.jit(_kernel_impl)
